# Initial kernel scaffold; baseline (speedup 1.0000x reference)
import functools

import jax
import jax.numpy as jnp
from jax import lax
from jax.experimental import pallas as pl
from jax.experimental.pallas import tpu as pltpu
from jax.experimental.pallas import tpu_sc as plsc

V = 1000
NFLAT = 51200
NC, NS, L = 2, 16, 16
NW = NC * NS
ROWS_PER_TILE = NFLAT // NW
CHUNK = 32
NCHUNKS = ROWS_PER_TILE // CHUNK


def _lse_body(tab_ref, out_ref):
    x = tab_ref[...]
    m = jnp.max(x, axis=1, keepdims=True)
    s = jnp.sum(jnp.exp(x - m), axis=1, keepdims=True)
    out_ref[...] = m + jnp.log(s)


def _table_lse(table):
    return pl.pallas_call(
        _lse_body,
        out_shape=jax.ShapeDtypeStruct((V, 1), jnp.float32),
    )(table)


_mesh = plsc.VectorSubcoreMesh(core_axis_name="c", subcore_axis_name="s")


@functools.partial(
    pl.kernel,
    mesh=_mesh,
    out_type=[
        jax.ShapeDtypeStruct((NFLAT, V), jnp.float32),
        jax.ShapeDtypeStruct((NW, L), jnp.float32),
    ],
    scratch_types=[
        pltpu.VMEM((ROWS_PER_TILE,), jnp.int32),
        pltpu.VMEM((ROWS_PER_TILE,), jnp.int32),
        pltpu.VMEM((V,), jnp.float32),
        pltpu.VMEM((CHUNK, V), jnp.float32),
        pltpu.VMEM((L,), jnp.float32),
        pltpu.SemaphoreType.DMA,
    ],
)
def _sc_gather(table_hbm, idxf_hbm, tgtf_hbm, lse_hbm,
               out_hbm, part_hbm,
               idx_v, tgt_v, lse_v, rows_v, part_v, sem):
    wid = lax.axis_index("s") * NC + lax.axis_index("c")
    base = wid * ROWS_PER_TILE
    pltpu.sync_copy(idxf_hbm.at[pl.ds(base, ROWS_PER_TILE)], idx_v)
    pltpu.sync_copy(tgtf_hbm.at[pl.ds(base, ROWS_PER_TILE)], tgt_v)
    pltpu.sync_copy(lse_hbm, lse_v)

    def chunk_body(c, acc):
        pltpu.async_copy(
            table_hbm.at[idx_v.at[pl.ds(c * CHUNK, CHUNK)]], rows_v, sem
        ).wait()
        for j in range(CHUNK // L):
            off = c * CHUNK + j * L
            idx_lanes = idx_v[pl.ds(off, L)]
            tgt_lanes = tgt_v[pl.ds(off, L)]
            row_ids = lax.iota(jnp.int32, L) + (j * L)
            tgt_logit = plsc.load_gather(rows_v, [row_ids, tgt_lanes])
            lse_vals = plsc.load_gather(lse_v, [idx_lanes])
            acc = acc + (lse_vals - tgt_logit)
        pltpu.sync_copy(rows_v, out_hbm.at[pl.ds(base + c * CHUNK, CHUNK)])
        return acc

    acc = lax.fori_loop(0, NCHUNKS, chunk_body,
                        jnp.zeros((L,), jnp.float32))
    part_v[...] = acc
    pltpu.sync_copy(part_v, part_hbm.at[wid])


def _loss_body(p_ref, out_ref):
    out_ref[...] = jnp.sum(p_ref[...]).reshape(1, 1) / NFLAT


def _loss_reduce(partials):
    return pl.pallas_call(
        _loss_body,
        out_shape=jax.ShapeDtypeStruct((1, 1), jnp.float32),
    )(partials)


def kernel(idx, targets, table):
    idx_flat = idx.astype(jnp.int32).reshape(NFLAT)
    tgt_flat = targets.astype(jnp.int32).reshape(NFLAT)
    table = table.astype(jnp.float32)
    lse = _table_lse(table).reshape(V)
    logits, partials = _sc_gather(table, idx_flat, tgt_flat, lse)
    loss = _loss_reduce(partials)[0, 0]
    return logits, loss


# speedup vs baseline: 1.3971x; 1.3971x over previous
import functools

import jax
import jax.numpy as jnp
from jax import lax
from jax.experimental import pallas as pl
from jax.experimental.pallas import tpu as pltpu
from jax.experimental.pallas import tpu_sc as plsc

V = 1000
NFLAT = 51200
NC, NS, L = 2, 16, 16
NW = NC * NS
ROWS_PER_TILE = NFLAT // NW
CHUNK = 32
NCHUNKS = ROWS_PER_TILE // CHUNK


def _lse_body(tab_ref, out_ref):
    x = tab_ref[...]
    m = jnp.max(x, axis=1, keepdims=True)
    s = jnp.sum(jnp.exp(x - m), axis=1, keepdims=True)
    out_ref[...] = m + jnp.log(s)


def _table_lse(table):
    return pl.pallas_call(
        _lse_body,
        out_shape=jax.ShapeDtypeStruct((V, 1), jnp.float32),
    )(table)


_mesh = plsc.VectorSubcoreMesh(core_axis_name="c", subcore_axis_name="s")


@functools.partial(
    pl.kernel,
    mesh=_mesh,
    compiler_params=pltpu.CompilerParams(
        use_tc_tiling_on_sc=False, needs_layout_passes=False),
    out_type=[
        jax.ShapeDtypeStruct((NFLAT, V), jnp.float32),
        jax.ShapeDtypeStruct((NW, L), jnp.float32),
    ],
    scratch_types=[
        pltpu.VMEM((ROWS_PER_TILE,), jnp.int32),
        pltpu.VMEM((ROWS_PER_TILE,), jnp.int32),
        pltpu.VMEM((V,), jnp.float32),
        pltpu.VMEM((CHUNK, V), jnp.float32),
        pltpu.VMEM((L,), jnp.float32),
        pltpu.SemaphoreType.DMA,
    ],
)
def _sc_gather(table_hbm, idxf_hbm, tgtf_hbm, lse_hbm,
               out_hbm, part_hbm,
               idx_v, tgt_v, lse_v, rows_v, part_v, sem):
    wid = lax.axis_index("s") * NC + lax.axis_index("c")
    base = wid * ROWS_PER_TILE
    pltpu.sync_copy(idxf_hbm.at[pl.ds(base, ROWS_PER_TILE)], idx_v)
    pltpu.sync_copy(tgtf_hbm.at[pl.ds(base, ROWS_PER_TILE)], tgt_v)
    pltpu.sync_copy(lse_hbm, lse_v)

    def chunk_body(c, acc):
        pltpu.async_copy(
            table_hbm.at[idx_v.at[pl.ds(c * CHUNK, CHUNK)]], rows_v, sem
        ).wait()
        for j in range(CHUNK // L):
            off = c * CHUNK + j * L
            idx_lanes = idx_v[pl.ds(off, L)]
            tgt_lanes = tgt_v[pl.ds(off, L)]
            row_ids = lax.iota(jnp.int32, L) + (j * L)
            tgt_logit = plsc.load_gather(rows_v, [row_ids, tgt_lanes])
            lse_vals = plsc.load_gather(lse_v, [idx_lanes])
            acc = acc + (lse_vals - tgt_logit)
        pltpu.sync_copy(rows_v, out_hbm.at[pl.ds(base + c * CHUNK, CHUNK)])
        return acc

    acc = lax.fori_loop(0, NCHUNKS, chunk_body,
                        jnp.zeros((L,), jnp.float32))
    part_v[...] = acc
    pltpu.sync_copy(part_v, part_hbm.at[wid])


def _loss_body(p_ref, out_ref):
    out_ref[...] = jnp.sum(p_ref[...]).reshape(1, 1) / NFLAT


def _loss_reduce(partials):
    return pl.pallas_call(
        _loss_body,
        out_shape=jax.ShapeDtypeStruct((1, 1), jnp.float32),
    )(partials)


def kernel(idx, targets, table):
    idx_flat = idx.astype(jnp.int32).reshape(NFLAT)
    tgt_flat = targets.astype(jnp.int32).reshape(NFLAT)
    table = table.astype(jnp.float32)
    lse = _table_lse(table).reshape(V)
    logits, partials = _sc_gather(table, idx_flat, tgt_flat, lse)
    loss = _loss_reduce(partials)[0, 0]
    return logits, loss


# speedup vs baseline: 1.4722x; 1.0537x over previous
import functools

import jax
import jax.numpy as jnp
from jax import lax
from jax.experimental import pallas as pl
from jax.experimental.pallas import tpu as pltpu
from jax.experimental.pallas import tpu_sc as plsc

V = 1000
NFLAT = 51200
NC, NS, L = 2, 16, 16
NW = NC * NS
ROWS_PER_TILE = NFLAT // NW
CHUNK = 32
NCHUNKS = ROWS_PER_TILE // CHUNK
NBUF = 3


def _lse_body(tab_ref, out_ref):
    x = tab_ref[...]
    m = jnp.max(x, axis=1, keepdims=True)
    s = jnp.sum(jnp.exp(x - m), axis=1, keepdims=True)
    out_ref[...] = m + jnp.log(s)


def _table_lse(table):
    return pl.pallas_call(
        _lse_body,
        out_shape=jax.ShapeDtypeStruct((V, 1), jnp.float32),
    )(table)


_mesh = plsc.VectorSubcoreMesh(core_axis_name="c", subcore_axis_name="s")


@functools.partial(
    pl.kernel,
    mesh=_mesh,
    compiler_params=pltpu.CompilerParams(
        use_tc_tiling_on_sc=False, needs_layout_passes=False),
    out_type=[
        jax.ShapeDtypeStruct((NFLAT, V), jnp.float32),
        jax.ShapeDtypeStruct((NW, L), jnp.float32),
    ],
    scratch_types=[
        pltpu.VMEM((ROWS_PER_TILE,), jnp.int32),
        pltpu.VMEM((ROWS_PER_TILE,), jnp.int32),
        pltpu.VMEM((V,), jnp.float32),
        [pltpu.VMEM((CHUNK, V), jnp.float32)] * NBUF,
        pltpu.VMEM((L,), jnp.float32),
        [pltpu.SemaphoreType.DMA] * NBUF,
        [pltpu.SemaphoreType.DMA] * NBUF,
    ],
)
def _sc_gather(table_hbm, idxf_hbm, tgtf_hbm, lse_hbm,
               out_hbm, part_hbm,
               idx_v, tgt_v, lse_v, rows_bufs, part_v, gsems, ssems):
    wid = lax.axis_index("s") * NC + lax.axis_index("c")
    base = wid * ROWS_PER_TILE
    pltpu.sync_copy(idxf_hbm.at[pl.ds(base, ROWS_PER_TILE)], idx_v)
    pltpu.sync_copy(tgtf_hbm.at[pl.ds(base, ROWS_PER_TILE)], tgt_v)
    pltpu.sync_copy(lse_hbm, lse_v)

    def start_gather(c, b):
        return pltpu.async_copy(
            table_hbm.at[idx_v.at[pl.ds(c * CHUNK, CHUNK)]],
            rows_bufs[b], gsems[b])

    def start_scatter(c, b):
        return pltpu.async_copy(
            rows_bufs[b], out_hbm.at[pl.ds(base + c * CHUNK, CHUNK)],
            ssems[b])

    acc = jnp.zeros((L,), jnp.float32)
    gh = [None] * NBUF
    sh = [None] * NBUF
    for b in range(NBUF - 1):
        gh[b] = start_gather(b, b)
    for c in range(NCHUNKS):
        b = c % NBUF
        gh[b].wait()
        for j in range(CHUNK // L):
            off = c * CHUNK + j * L
            idx_lanes = idx_v[pl.ds(off, L)]
            tgt_lanes = tgt_v[pl.ds(off, L)]
            row_ids = lax.iota(jnp.int32, L) + (j * L)
            tgt_logit = plsc.load_gather(rows_bufs[b], [row_ids, tgt_lanes])
            lse_vals = plsc.load_gather(lse_v, [idx_lanes])
            acc = acc + (lse_vals - tgt_logit)
        sh[b] = start_scatter(c, b)
        nc = c + NBUF - 1
        if nc < NCHUNKS:
            nb = nc % NBUF
            if sh[nb] is not None:
                sh[nb].wait()
                sh[nb] = None
            gh[nb] = start_gather(nc, nb)
    for b in range(NBUF):
        if sh[b] is not None:
            sh[b].wait()
    part_v[...] = acc
    pltpu.sync_copy(part_v, part_hbm.at[wid])


def _loss_body(p_ref, out_ref):
    out_ref[...] = jnp.sum(p_ref[...]).reshape(1, 1) / NFLAT


def _loss_reduce(partials):
    return pl.pallas_call(
        _loss_body,
        out_shape=jax.ShapeDtypeStruct((1, 1), jnp.float32),
    )(partials)


def kernel(idx, targets, table):
    idx_flat = idx.astype(jnp.int32).reshape(NFLAT)
    tgt_flat = targets.astype(jnp.int32).reshape(NFLAT)
    table = table.astype(jnp.float32)
    lse = _table_lse(table).reshape(V)
    logits, partials = _sc_gather(table, idx_flat, tgt_flat, lse)
    loss = _loss_reduce(partials)[0, 0]
    return logits, loss
